# baseline (speedup 1.0000x reference)
import functools

import jax
import jax.numpy as jnp
from jax.experimental import pallas as pl
from jax.experimental.pallas import tpu as pltpu

TOPK = 5
MIN_SIM = 0.2
NEG = -3.0
QB = 256
XB = 2048


def _sim_kernel(q_ref, x_ref, ids_ref, ss_ref, res_ref, rs_ref, ri_ref,
                *, nx, n_valid):
    qi = pl.program_id(0)
    xi = pl.program_id(1)

    @pl.when(xi == 0)
    def _init():
        rs_ref[...] = jnp.full_like(rs_ref, NEG)
        ri_ref[...] = jnp.full_like(ri_ref, -1.0)

    q = q_ref[...]
    x = x_ref[...]
    s = jax.lax.dot_general(q, x, (((1,), (1,)), ((), ())),
                            preferred_element_type=jnp.float32,
                            precision=jax.lax.Precision.DEFAULT)

    col = jax.lax.broadcasted_iota(jnp.int32, (QB, XB), 1)
    gcol = col + xi * XB
    s = jnp.where(gcol < n_valid, s, NEG)

    ids_row = ids_ref[0, 0, :]

    big = jnp.int32(XB + 1)
    for _ in range(TOPK):
        m = jnp.max(s, axis=1, keepdims=True)
        eq = s == m
        idx = jnp.min(jnp.where(eq, col, big), axis=1, keepdims=True)
        hit = col == idx
        cid = jnp.max(jnp.where(hit, ids_row[None, :], -1.0),
                      axis=1, keepdims=True)
        s = jnp.where(hit, NEG, s)
        tv, ti = m, cid
        for j in range(TOPK):
            cur_v = rs_ref[:, j:j + 1]
            cur_i = ri_ref[:, j:j + 1]
            gt = tv > cur_v
            rs_ref[:, j:j + 1] = jnp.where(gt, tv, cur_v)
            ri_ref[:, j:j + 1] = jnp.where(gt, ti, cur_i)
            tv = jnp.where(gt, cur_v, tv)
            ti = jnp.where(gt, cur_i, ti)

    @pl.when(xi == nx - 1)
    def _epilogue():
        sims = [rs_ref[:, j:j + 1] for j in range(TOPK)]
        ids = [ri_ref[:, j:j + 1] for j in range(TOPK)]
        mask = [sv >= MIN_SIM for sv in sims]
        zero = jnp.zeros((QB, 1), jnp.float32)
        counts = []
        for i in range(TOPK):
            c = zero
            for j in range(TOPK):
                c = c + jnp.where(mask[j] & (ids[i] == ids[j]), 1.0, 0.0)
            counts.append(c)
        maxc = zero
        for i in range(TOPK):
            maxc = jnp.maximum(maxc, jnp.where(mask[i], counts[i], 0.0))
        bid = jnp.full((QB, 1), 1e9, jnp.float32)
        for i in range(TOPK):
            sel = mask[i] & (counts[i] == maxc)
            bid = jnp.minimum(bid, jnp.where(sel, ids[i], 1e9))
        resf = jnp.where(maxc > 0, bid, -1.0)
        ss = zero
        for i in range(TOPK):
            sel = mask[i] & (ids[i] == resf)
            ss = jnp.maximum(ss, jnp.where(sel, sims[i], 0.0))
        ss_ref[...] = ss[:, 0]
        res_ref[...] = resf[:, 0].astype(jnp.int32)


@jax.jit
def _run(descriptors, xmat, ids_blocks):
    nq = descriptors.shape[0] // QB
    nx = ids_blocks.shape[0]
    n_valid = xmat.shape[0]
    grid = (nq, nx)
    kfn = functools.partial(_sim_kernel, nx=nx, n_valid=n_valid)
    ss, res = pl.pallas_call(
        kfn,
        grid=grid,
        in_specs=[
            pl.BlockSpec((QB, 128), lambda qi, xi: (qi, 0)),
            pl.BlockSpec((XB, 128), lambda qi, xi: (xi, 0)),
            pl.BlockSpec((1, 1, XB), lambda qi, xi: (xi, 0, 0)),
        ],
        out_specs=[
            pl.BlockSpec((QB,), lambda qi, xi: (qi,)),
            pl.BlockSpec((QB,), lambda qi, xi: (qi,)),
        ],
        out_shape=[
            jax.ShapeDtypeStruct((descriptors.shape[0],), jnp.float32),
            jax.ShapeDtypeStruct((descriptors.shape[0],), jnp.int32),
        ],
        scratch_shapes=[
            pltpu.VMEM((QB, 8), jnp.float32),
            pltpu.VMEM((QB, 8), jnp.float32),
        ],
    )(descriptors, xmat, ids_blocks)
    return ss, res


def kernel(final_boxes, descriptors, places_db):
    xmat = places_db[:, :-1]
    ids = places_db[:, -1]
    n = xmat.shape[0]
    nx = pl.cdiv(n, XB)
    ids_pad = jnp.pad(ids, (0, nx * XB - n)).reshape(nx, 1, XB)
    ss, res = _run(descriptors, xmat, ids_pad)
    return (final_boxes, ss, res)


# speedup vs baseline: 1.3180x; 1.3180x over previous
import functools

import jax
import jax.numpy as jnp
from jax.experimental import pallas as pl
from jax.experimental.pallas import tpu as pltpu

TOPK = 5
MIN_SIM = 0.2
THRESH = 0.19
NEG = -3.0
QB = 256
XB = 2048
IDB = 9


def _sim_kernel(q_ref, x_ref, ids_ref, ss_ref, res_ref, s_ref, rs_ref, rk_ref,
                *, nx, nq, n_valid):
    xi = pl.program_id(0)
    qi = pl.program_id(1)

    @pl.when(xi == 0)
    def _init():
        rs_ref[qi] = jnp.full((8, QB), NEG, jnp.float32)
        rk_ref[qi] = jnp.zeros((8, QB), jnp.int32)

    q = q_ref[...]
    x = x_ref[...]
    s = jax.lax.dot_general(q, x, (((1,), (1,)), ((), ())),
                            preferred_element_type=jnp.float32,
                            precision=jax.lax.Precision.DEFAULT)

    @pl.when(xi == nx - 1)
    def _mask_tail():
        col = jax.lax.broadcasted_iota(jnp.int32, (QB, XB), 1)
        s_ref[...] = jnp.where(col + xi * XB < n_valid, s, NEG)

    @pl.when(xi != nx - 1)
    def _store_s():
        s_ref[...] = s

    cols = jax.lax.iota(jnp.int32, XB)
    kb = (cols << IDB) | ids_ref[0, 0, :].astype(jnp.int32)
    kb2 = kb[None, :]
    big = jnp.int32(XB << IDB)

    t = jnp.maximum(rs_ref[qi, TOPK - 1], THRESH)
    sv = s_ref[...]
    cnt = jnp.sum((sv > t[:, None]).astype(jnp.float32), axis=1)
    n_it = jnp.minimum(jnp.max(cnt), float(TOPK)).astype(jnp.int32)

    def body(_, carry):
        scur = s_ref[...]
        m = jnp.max(scur, axis=1, keepdims=True)
        key = jnp.min(jnp.where(scur == m, kb2, big),
                      axis=1, keepdims=True)
        s_ref[...] = jnp.where(kb2 == key, NEG, scur)
        tv = jnp.reshape(m, (QB,))
        tk = jnp.reshape(key, (QB,))
        for j in range(TOPK):
            cur = rs_ref[qi, j]
            curk = rk_ref[qi, j]
            gt = tv > cur
            rs_ref[qi, j] = jnp.maximum(tv, cur)
            rk_ref[qi, j] = jnp.where(gt, tk, curk)
            tv = jnp.minimum(tv, cur)
            tk = jnp.where(gt, curk, tk)
        return carry

    jax.lax.fori_loop(0, n_it, body, 0)

    @pl.when(xi == nx - 1)
    def _epilogue():
        sims = [rs_ref[qi, j] for j in range(TOPK)]
        ids = [(rk_ref[qi, j] & ((1 << IDB) - 1)).astype(jnp.float32)
               for j in range(TOPK)]
        mask = [sv2 >= MIN_SIM for sv2 in sims]
        zero = jnp.zeros((QB,), jnp.float32)
        counts = []
        for i in range(TOPK):
            c = zero
            for j in range(TOPK):
                c = c + jnp.where(mask[j] & (ids[i] == ids[j]), 1.0, 0.0)
            counts.append(c)
        maxc = zero
        for i in range(TOPK):
            maxc = jnp.maximum(maxc, jnp.where(mask[i], counts[i], 0.0))
        bid = jnp.full((QB,), 1e9, jnp.float32)
        for i in range(TOPK):
            sel = mask[i] & (counts[i] == maxc)
            bid = jnp.minimum(bid, jnp.where(sel, ids[i], 1e9))
        resf = jnp.where(maxc > 0, bid, -1.0)
        ss = zero
        for i in range(TOPK):
            sel = mask[i] & (ids[i] == resf)
            ss = jnp.maximum(ss, jnp.where(sel, sims[i], 0.0))
        ss_ref[...] = ss
        res_ref[...] = resf.astype(jnp.int32)


@jax.jit
def _run(descriptors, xmat, ids_blocks):
    nq = descriptors.shape[0] // QB
    nx = ids_blocks.shape[0]
    n_valid = xmat.shape[0]
    grid = (nx, nq)
    kfn = functools.partial(_sim_kernel, nx=nx, nq=nq, n_valid=n_valid)
    ss, res = pl.pallas_call(
        kfn,
        grid=grid,
        in_specs=[
            pl.BlockSpec((QB, 128), lambda xi, qi: (qi, 0)),
            pl.BlockSpec((XB, 128), lambda xi, qi: (xi, 0)),
            pl.BlockSpec((1, 1, XB), lambda xi, qi: (xi, 0, 0)),
        ],
        out_specs=[
            pl.BlockSpec((QB,), lambda xi, qi: (qi,)),
            pl.BlockSpec((QB,), lambda xi, qi: (qi,)),
        ],
        out_shape=[
            jax.ShapeDtypeStruct((descriptors.shape[0],), jnp.float32),
            jax.ShapeDtypeStruct((descriptors.shape[0],), jnp.int32),
        ],
        scratch_shapes=[
            pltpu.VMEM((QB, XB), jnp.float32),
            pltpu.VMEM((4, 8, QB), jnp.float32),
            pltpu.VMEM((4, 8, QB), jnp.int32),
        ],
    )(descriptors, xmat, ids_blocks)
    return ss, res


def kernel(final_boxes, descriptors, places_db):
    xmat = places_db[:, :-1]
    ids = places_db[:, -1]
    n = xmat.shape[0]
    nx = pl.cdiv(n, XB)
    ids_pad = jnp.pad(ids, (0, nx * XB - n)).reshape(nx, 1, XB)
    ss, res = _run(descriptors, xmat, ids_pad)
    return (final_boxes, ss, res)
